# baseline (speedup 1.0000x reference)
import functools

import jax
import jax.numpy as jnp
from jax import lax
from jax.experimental import pallas as pl
from jax.experimental.pallas import tpu as pltpu
from jax.experimental.pallas import tpu_sc as plsc

B = 1024
S = 512
VOCAB = 1000000
EMB = 300
HID = 16
NPROJ = 128
NCHUNK = 4
G = S // NCHUNK
ROWS_BLK = 8192


def _proj_body(t_ref, w_ref, b_ref, o_ref):
  t = t_ref[...].astype(jnp.bfloat16)
  w = w_ref[...].astype(jnp.bfloat16)
  o_ref[...] = lax.dot_general(
      t, w, (((0,), (0,)), ((), ())),
      preferred_element_type=jnp.float32) + b_ref[...]


def _make_sc_head(num_cores, num_workers):
  b_per_w = B // num_workers
  mesh = plsc.VectorSubcoreMesh(core_axis_name="c", subcore_axis_name="s")

  @functools.partial(
      pl.kernel,
      out_type=jax.ShapeDtypeStruct((B,), jnp.float32),
      mesh=mesh,
      scratch_types=[
          pltpu.VMEM((b_per_w * NCHUNK, G), jnp.int32),
          pltpu.VMEM((G, NPROJ), jnp.float32),
          pltpu.VMEM((G, NPROJ), jnp.float32),
          pltpu.VMEM((b_per_w,), jnp.float32),
          pltpu.VMEM((2, HID), jnp.float32),
          pltpu.SemaphoreType.DMA,
          pltpu.SemaphoreType.DMA,
      ],
      compiler_params=pltpu.CompilerParams(needs_layout_passes=False,
                                           use_tc_tiling_on_sc=True),
  )
  def sc_head(idx_hbm, proj_hbm, wb_hbm, out_hbm, idx_v, rows0, rows1,
              stage, wbv, sem0, sem1):
    cid = lax.axis_index("c")
    sid = lax.axis_index("s")
    wid = sid * num_cores + cid
    base = wid * b_per_w
    pltpu.sync_copy(idx_hbm.at[pl.ds(base * NCHUNK, b_per_w * NCHUNK)], idx_v)
    pltpu.sync_copy(wb_hbm, wbv)

    rows = (rows0, rows1)
    sems = (sem0, sem1)

    def start(i, k, buf):
      pltpu.async_copy(proj_hbm.at[idx_v.at[i * NCHUNK + k]], rows[buf],
                       sems[buf])

    def wait(buf):
      pltpu.make_async_copy(proj_hbm.at[idx_v.at[0]], rows[buf],
                            sems[buf]).wait()

    start(0, 0, 0)
    w2reg = wbv[0, pl.ds(0, HID)]
    b2reg = wbv[1, pl.ds(0, HID)]
    lane = lax.iota(jnp.int32, 16)

    def make_batch_body(g):
      def batch_body(i, zvec):
        gi = g * 16 + i
        acc = jnp.zeros((16,), jnp.float32)
        for k in range(NCHUNK):
          if k + 1 < NCHUNK:
            start(gi, k + 1, (k + 1) % 2)
          else:
            start(jnp.minimum(gi + 1, b_per_w - 1), 0, 0)
          wait(k % 2)
          rb = rows[k % 2]

          def row_body(r, a):
            return a + rb[r, pl.ds(0, HID)]

          acc = lax.fori_loop(0, G, row_body, acc, unroll=8)

        h = jnp.maximum(acc * (1.0 / S), 0.0)
        z = jnp.sum(h * w2reg)
        return jnp.where(lane == i, z, zvec)
      return batch_body

    for g in range(b_per_w // 16):
      zvec = lax.fori_loop(0, 16, make_batch_body(g),
                           jnp.zeros((16,), jnp.float32), unroll=False)
      stage[pl.ds(g * 16, 16)] = 1.0 / (
          1.0 + jnp.exp(-(zvec + b2reg)))

    wait(0)
    pltpu.sync_copy(stage, out_hbm.at[pl.ds(base, b_per_w)])

  return sc_head


def kernel(indices, table, W1, b1, W2, b2):
  info = plsc.get_sparse_core_info()
  num_workers = info.num_cores * info.num_subcores

  w1p = jnp.pad(W1, ((0, 0), (0, NPROJ - HID)))
  b1p = jnp.pad(b1, (0, NPROJ - HID)).reshape(1, NPROJ)
  proj = pl.pallas_call(
      _proj_body,
      grid=(pl.cdiv(VOCAB, ROWS_BLK),),
      in_specs=[
          pl.BlockSpec((EMB, ROWS_BLK), lambda i: (0, i)),
          pl.BlockSpec((EMB, NPROJ), lambda i: (0, 0)),
          pl.BlockSpec((1, NPROJ), lambda i: (0, 0)),
      ],
      out_specs=pl.BlockSpec((ROWS_BLK, NPROJ), lambda i: (i, 0)),
      out_shape=jax.ShapeDtypeStruct((VOCAB, NPROJ), jnp.float32),
  )(table.T, w1p, b1p)

  idx2 = indices.reshape(B * NCHUNK, G)
  wb = jnp.stack([W2.reshape(HID), jnp.broadcast_to(b2, (HID,))])
  out = _make_sc_head(info.num_cores, num_workers)(idx2, proj, wb)
  return out.reshape(B, 1)


# speedup vs baseline: 1.0224x; 1.0224x over previous
import functools

import jax
import jax.numpy as jnp
from jax import lax
from jax.experimental import pallas as pl
from jax.experimental.pallas import tpu as pltpu
from jax.experimental.pallas import tpu_sc as plsc

B = 1024
S = 512
VOCAB = 1000000
EMB = 300
HID = 16
NPROJ = 128
NCHUNK = 4
G = S // NCHUNK
ROWS_BLK = 8192


def _proj_body(t_ref, w_ref, b_ref, o_ref):
  t = t_ref[...].astype(jnp.bfloat16)
  w = w_ref[...].astype(jnp.bfloat16)
  o_ref[...] = lax.dot_general(
      t, w, (((0,), (0,)), ((), ())),
      preferred_element_type=jnp.float32) + b_ref[...]


def _make_sc_head(num_cores, num_workers):
  b_per_w = B // num_workers
  mesh = plsc.VectorSubcoreMesh(core_axis_name="c", subcore_axis_name="s")

  @functools.partial(
      pl.kernel,
      out_type=jax.ShapeDtypeStruct((B,), jnp.float32),
      mesh=mesh,
      scratch_types=[
          pltpu.VMEM((b_per_w * NCHUNK, G), jnp.int32),
          pltpu.VMEM((G, NPROJ), jnp.float32),
          pltpu.VMEM((G, NPROJ), jnp.float32),
          pltpu.VMEM((G, NPROJ), jnp.float32),
          pltpu.VMEM((G, NPROJ), jnp.float32),
          pltpu.VMEM((b_per_w,), jnp.float32),
          pltpu.VMEM((2, HID), jnp.float32),
          pltpu.SemaphoreType.DMA,
          pltpu.SemaphoreType.DMA,
          pltpu.SemaphoreType.DMA,
          pltpu.SemaphoreType.DMA,
      ],
      compiler_params=pltpu.CompilerParams(needs_layout_passes=False,
                                           use_tc_tiling_on_sc=True),
  )
  def sc_head(idx_hbm, proj_hbm, wb_hbm, out_hbm, idx_v, rows0, rows1,
              rows2, rows3, stage, wbv, sem0, sem1, sem2, sem3):
    cid = lax.axis_index("c")
    sid = lax.axis_index("s")
    wid = sid * num_cores + cid
    base = wid * b_per_w
    pltpu.sync_copy(idx_hbm.at[pl.ds(base * NCHUNK, b_per_w * NCHUNK)], idx_v)
    pltpu.sync_copy(wb_hbm, wbv)

    rows = (rows0, rows1, rows2, rows3)
    sems = (sem0, sem1, sem2, sem3)

    def start(i, k, buf):
      pltpu.async_copy(proj_hbm.at[idx_v.at[i * NCHUNK + k]], rows[buf],
                       sems[buf])

    def wait(buf):
      pltpu.make_async_copy(proj_hbm.at[idx_v.at[0]], rows[buf],
                            sems[buf]).wait()

    start(0, 0, 0)
    start(0, 1, 1)
    w2reg = wbv[0, pl.ds(0, HID)]
    b2reg = wbv[1, pl.ds(0, HID)]
    lane = lax.iota(jnp.int32, 16)

    def make_batch_body(g):
      def batch_body(i, zvec):
        gi = g * 16 + i
        acc = jnp.zeros((16,), jnp.float32)
        for k in range(NCHUNK):
          k2 = (k + 2) % NCHUNK
          i2 = jnp.minimum(gi + (k + 2) // NCHUNK, b_per_w - 1)
          start(i2, k2, k2)
          wait(k)
          rb = rows[k]

          def row_body(r, a):
            return a + rb[r, pl.ds(0, HID)]

          acc = lax.fori_loop(0, G, row_body, acc, unroll=8)

        h = jnp.maximum(acc * (1.0 / S), 0.0)
        z = jnp.sum(h * w2reg)
        return jnp.where(lane == i, z, zvec)
      return batch_body

    for g in range(b_per_w // 16):
      zvec = lax.fori_loop(0, 16, make_batch_body(g),
                           jnp.zeros((16,), jnp.float32), unroll=False)
      stage[pl.ds(g * 16, 16)] = 1.0 / (
          1.0 + jnp.exp(-(zvec + b2reg)))

    wait(0)
    wait(1)
    pltpu.sync_copy(stage, out_hbm.at[pl.ds(base, b_per_w)])

  return sc_head


def kernel(indices, table, W1, b1, W2, b2):
  info = plsc.get_sparse_core_info()
  num_workers = info.num_cores * info.num_subcores

  w1p = jnp.pad(W1, ((0, 0), (0, NPROJ - HID)))
  b1p = jnp.pad(b1, (0, NPROJ - HID)).reshape(1, NPROJ)
  proj = pl.pallas_call(
      _proj_body,
      grid=(pl.cdiv(VOCAB, ROWS_BLK),),
      in_specs=[
          pl.BlockSpec((EMB, ROWS_BLK), lambda i: (0, i)),
          pl.BlockSpec((EMB, NPROJ), lambda i: (0, 0)),
          pl.BlockSpec((1, NPROJ), lambda i: (0, 0)),
      ],
      out_specs=pl.BlockSpec((ROWS_BLK, NPROJ), lambda i: (i, 0)),
      out_shape=jax.ShapeDtypeStruct((VOCAB, NPROJ), jnp.float32),
  )(table.T, w1p, b1p)

  idx2 = indices.reshape(B * NCHUNK, G)
  wb = jnp.stack([W2.reshape(HID), jnp.broadcast_to(b2, (HID,))])
  out = _make_sc_head(info.num_cores, num_workers)(idx2, proj, wb)
  return out.reshape(B, 1)
